# Initial kernel scaffold; baseline (speedup 1.0000x reference)
import jax
import jax.numpy as jnp
from jax.experimental import pallas as pl


def kernel(x, indptr):
    raise NotImplementedError("write your pallas kernel here")


# speedup vs baseline: 16.2128x; 16.2128x over previous
import functools

import jax
import jax.numpy as jnp
from jax import lax
from jax.experimental import pallas as pl
from jax.experimental.pallas import tpu as pltpu
from jax.experimental.pallas import tpu_sc as plsc

N = 320000
S = 10000
D = 128
NLANE = 16
NVEC = D // NLANE

NW = 32
SEG_W = 320
S_PAD = NW * SEG_W
IPTR_BUF = 344
IPTR_PAD = (NW - 1) * SEG_W + IPTR_BUF
CHUNK = 256


def _ld(ref, i):
    return ref[pl.ds(i, NLANE)][0]


def _seg_kernel(x_hbm, iptr_hbm, out_hbm, iptr_v, buf, out_v):
    wid = lax.axis_index("s") * 2 + lax.axis_index("c")
    s0 = pl.multiple_of(wid * SEG_W, 8)

    pltpu.sync_copy(iptr_hbm.at[pl.ds(s0, IPTR_BUF)], iptr_v)

    zero = jnp.zeros((NLANE,), jnp.float32)

    r0 = _ld(iptr_v, 0)
    r_end = _ld(iptr_v, SEG_W)
    base0 = (r0 // 8) * 8
    n_chunks = (r_end - base0 + CHUNK - 1) // CHUNK

    def body(t, carry):
        r_cur, s, staged_win, *acc = carry
        win = (r_cur - base0) // CHUNK
        win_start = base0 + win * CHUNK
        start = pl.multiple_of(jnp.minimum(win_start, N - CHUNK), 8)
        win_end = win_start + CHUNK

        @pl.when(win != staged_win)
        def _stage():
            pltpu.sync_copy(x_hbm.at[pl.ds(start, CHUNK)], buf)

        seg_end = _ld(iptr_v, s + 1)
        e = jnp.minimum(jnp.minimum(seg_end, win_end), r_end)

        def row_body(i, ac):
            idx = r_cur + i - start
            return tuple(
                ac[j] + buf[idx, pl.ds(j * NLANE, NLANE)]
                for j in range(NVEC)
            )

        acc = lax.fori_loop(0, e - r_cur, row_body, tuple(acc))

        finished = jnp.logical_and(e >= seg_end, s < SEG_W)

        @pl.when(finished)
        def _flush():
            for j in range(NVEC):
                out_v[s, pl.ds(j * NLANE, NLANE)] = acc[j]

        s_next = s + jnp.where(finished, 1, 0)
        acc_next = tuple(jnp.where(finished, zero, a) for a in acc)
        return (e, s_next, win) + acc_next

    init = (r0, jnp.int32(0), jnp.int32(-1)) + tuple(zero for _ in range(NVEC))
    lax.fori_loop(0, n_chunks + SEG_W, body, init)

    pltpu.sync_copy(out_v, out_hbm.at[pl.ds(s0, SEG_W)])


@jax.jit
def _run(x, iptr_pad):
    mesh = plsc.VectorSubcoreMesh(core_axis_name="c", subcore_axis_name="s")
    f = functools.partial(
        pl.kernel,
        mesh=mesh,
        out_type=jax.ShapeDtypeStruct((S_PAD, D), jnp.float32),
        scratch_types=[
            pltpu.VMEM((IPTR_BUF,), jnp.int32),
            pltpu.VMEM((CHUNK, D), jnp.float32),
            pltpu.VMEM((SEG_W, D), jnp.float32),
        ],
    )(_seg_kernel)
    return f(x, iptr_pad)


def kernel(x, indptr):
    pad = jnp.full((IPTR_PAD - (S + 1),), N, dtype=indptr.dtype)
    iptr_pad = jnp.concatenate([indptr, pad])
    out = _run(x, iptr_pad)
    return out[:S]


# speedup vs baseline: 25.6214x; 1.5803x over previous
import functools

import jax
import jax.numpy as jnp
from jax import lax
from jax.experimental import pallas as pl
from jax.experimental.pallas import tpu as pltpu
from jax.experimental.pallas import tpu_sc as plsc

N = 320000
S = 10000
D = 128
NLANE = 16
NVEC = D // NLANE

NW = 32
SEG_W = 320
S_PAD = NW * SEG_W
IPTR_BUF = 344
IPTR_PAD = (NW - 1) * SEG_W + IPTR_BUF
CHUNK = 256


def _ld(ref, i):
    return ref[pl.ds(i, NLANE)][0]


def _seg_kernel(x_hbm, iptr_hbm, out_hbm, iptr_v, buf, out_v, sem0, sem1):
    wid = lax.axis_index("s") * 2 + lax.axis_index("c")
    s0 = pl.multiple_of(wid * SEG_W, 8)

    pltpu.sync_copy(iptr_hbm.at[pl.ds(s0, IPTR_BUF)], iptr_v)

    zero = jnp.zeros((NLANE,), jnp.float32)

    r0 = _ld(iptr_v, 0)
    r_end = _ld(iptr_v, SEG_W)
    base0 = (r0 // 8) * 8
    n_chunks = (r_end - base0 + CHUNK - 1) // CHUNK

    def start_of(win):
        return pl.multiple_of(
            jnp.minimum(base0 + win * CHUNK, N - CHUNK), 8)

    @pl.when(n_chunks > 0)
    def _prime():
        pltpu.async_copy(x_hbm.at[pl.ds(start_of(0), CHUNK)], buf.at[0], sem0)

    def body(t, carry):
        r_cur, s, staged_win, *acc = carry
        win = (r_cur - base0) // CHUNK
        start = start_of(win)
        win_end = base0 + (win + 1) * CHUNK
        p = win % 2

        @pl.when(jnp.logical_and(win != staged_win, win < n_chunks))
        def _advance():
            src = x_hbm.at[pl.ds(start, CHUNK)]

            @pl.when(p == 0)
            def _():
                pltpu.make_async_copy(src, buf.at[0], sem0).wait()

            @pl.when(p == 1)
            def _():
                pltpu.make_async_copy(src, buf.at[1], sem1).wait()

            @pl.when(win + 1 < n_chunks)
            def _prefetch():
                nsrc = x_hbm.at[pl.ds(start_of(win + 1), CHUNK)]

                @pl.when(p == 0)
                def _():
                    pltpu.async_copy(nsrc, buf.at[1], sem1)

                @pl.when(p == 1)
                def _():
                    pltpu.async_copy(nsrc, buf.at[0], sem0)

        seg_end = _ld(iptr_v, s + 1)
        e = jnp.minimum(jnp.minimum(seg_end, win_end), r_end)
        n_rows = e - r_cur

        def row_pair(k, ac):
            idx = r_cur + 2 * k - start
            for j in range(NVEC):
                ac = ac[:j] + (ac[j] + buf[p, idx, pl.ds(j * NLANE, NLANE)],) + ac[j + 1:]
            for j in range(NVEC):
                ac = ac[:j] + (ac[j] + buf[p, idx + 1, pl.ds(j * NLANE, NLANE)],) + ac[j + 1:]
            return ac

        def row_one(i, ac):
            idx = r_cur + i - start
            return tuple(
                ac[j] + buf[p, idx, pl.ds(j * NLANE, NLANE)]
                for j in range(NVEC)
            )

        acc = lax.fori_loop(0, n_rows // 2, row_pair, tuple(acc))
        acc = lax.fori_loop(n_rows // 2 * 2, n_rows, row_one, acc)

        finished = jnp.logical_and(e >= seg_end, s < SEG_W)

        @pl.when(finished)
        def _flush():
            for j in range(NVEC):
                out_v[s, pl.ds(j * NLANE, NLANE)] = acc[j]

        s_next = s + jnp.where(finished, 1, 0)
        acc_next = tuple(jnp.where(finished, zero, a) for a in acc)
        return (e, s_next, win) + acc_next

    init = (r0, jnp.int32(0), jnp.int32(-1)) + tuple(zero for _ in range(NVEC))
    lax.fori_loop(0, n_chunks + SEG_W, body, init)

    pltpu.sync_copy(out_v, out_hbm.at[pl.ds(s0, SEG_W)])


@jax.jit
def _run(x, iptr_pad):
    mesh = plsc.VectorSubcoreMesh(core_axis_name="c", subcore_axis_name="s")
    f = functools.partial(
        pl.kernel,
        mesh=mesh,
        out_type=jax.ShapeDtypeStruct((S_PAD, D), jnp.float32),
        scratch_types=[
            pltpu.VMEM((IPTR_BUF,), jnp.int32),
            pltpu.VMEM((2, CHUNK, D), jnp.float32),
            pltpu.VMEM((SEG_W, D), jnp.float32),
            pltpu.SemaphoreType.DMA,
            pltpu.SemaphoreType.DMA,
        ],
    )(_seg_kernel)
    return f(x, iptr_pad)


def kernel(x, indptr):
    pad = jnp.full((IPTR_PAD - (S + 1),), N, dtype=indptr.dtype)
    iptr_pad = jnp.concatenate([indptr, pad])
    out = _run(x, iptr_pad)
    return out[:S]


# speedup vs baseline: 26.6205x; 1.0390x over previous
import functools

import jax
import jax.numpy as jnp
from jax import lax
from jax.experimental import pallas as pl
from jax.experimental.pallas import tpu as pltpu
from jax.experimental.pallas import tpu_sc as plsc

N = 320000
S = 10000
D = 128
NLANE = 16
NVEC = D // NLANE

NW = 32
SEG_W = 320
SEG_LAST = S - (NW - 1) * SEG_W
IPTR_BUF = 344
IPTR_PAD = (NW - 1) * SEG_W + IPTR_BUF
CHUNK = 256


def _ld(ref, i):
    return ref[pl.ds(i, NLANE)][0]


def _seg_kernel(x_hbm, iptr_hbm, out_hbm, iptr_v, buf, out_v, sem0, sem1):
    wid = lax.axis_index("s") * 2 + lax.axis_index("c")
    s0 = pl.multiple_of(wid * SEG_W, 8)

    pltpu.sync_copy(iptr_hbm.at[pl.ds(s0, IPTR_BUF)], iptr_v)

    zero = jnp.zeros((NLANE,), jnp.float32)

    nseg = jnp.where(wid == NW - 1, SEG_LAST, SEG_W)
    r0 = _ld(iptr_v, 0)
    r_end = _ld(iptr_v, nseg)
    base0 = (r0 // 8) * 8
    n_chunks = (r_end - base0 + CHUNK - 1) // CHUNK

    def start_of(win):
        return pl.multiple_of(
            jnp.minimum(base0 + win * CHUNK, N - CHUNK), 8)

    @pl.when(n_chunks > 0)
    def _prime():
        pltpu.async_copy(x_hbm.at[pl.ds(start_of(0), CHUNK)], buf.at[0], sem0)

    def body(t, carry):
        r_cur, s, staged_win, *acc = carry
        win = (r_cur - base0) // CHUNK
        start = start_of(win)
        win_end = base0 + (win + 1) * CHUNK
        p = win % 2

        @pl.when(jnp.logical_and(win != staged_win, win < n_chunks))
        def _advance():
            src = x_hbm.at[pl.ds(start, CHUNK)]

            @pl.when(p == 0)
            def _():
                pltpu.make_async_copy(src, buf.at[0], sem0).wait()

            @pl.when(p == 1)
            def _():
                pltpu.make_async_copy(src, buf.at[1], sem1).wait()

            @pl.when(win + 1 < n_chunks)
            def _prefetch():
                nsrc = x_hbm.at[pl.ds(start_of(win + 1), CHUNK)]

                @pl.when(p == 0)
                def _():
                    pltpu.async_copy(nsrc, buf.at[1], sem1)

                @pl.when(p == 1)
                def _():
                    pltpu.async_copy(nsrc, buf.at[0], sem0)

        seg_end = _ld(iptr_v, s + 1)
        e = jnp.minimum(jnp.minimum(seg_end, win_end), r_end)
        n_rows = e - r_cur

        def row_quad(k, ac):
            idx = r_cur + 4 * k - start
            ac = list(ac)
            for u in range(4):
                for j in range(NVEC):
                    ac[j] = ac[j] + buf[p, idx + u, pl.ds(j * NLANE, NLANE)]
            return tuple(ac)

        def row_one(i, ac):
            idx = r_cur + i - start
            return tuple(
                ac[j] + buf[p, idx, pl.ds(j * NLANE, NLANE)]
                for j in range(NVEC)
            )

        acc = lax.fori_loop(0, n_rows // 4, row_quad, tuple(acc))
        acc = lax.fori_loop(n_rows // 4 * 4, n_rows, row_one, acc)

        finished = jnp.logical_and(e >= seg_end, s < nseg)

        @pl.when(finished)
        def _flush():
            for j in range(NVEC):
                out_v[s, pl.ds(j * NLANE, NLANE)] = acc[j]

        s_next = s + jnp.where(finished, 1, 0)
        acc_next = tuple(jnp.where(finished, zero, a) for a in acc)
        return (e, s_next, win) + acc_next

    init = (r0, jnp.int32(0), jnp.int32(-1)) + tuple(zero for _ in range(NVEC))
    lax.fori_loop(0, n_chunks + SEG_W, body, init)

    @pl.when(wid < NW - 1)
    def _store_full():
        pltpu.sync_copy(out_v, out_hbm.at[pl.ds(s0, SEG_W)])

    @pl.when(wid == NW - 1)
    def _store_last():
        pltpu.sync_copy(out_v.at[pl.ds(0, SEG_LAST)],
                        out_hbm.at[pl.ds(s0, SEG_LAST)])


@jax.jit
def _run(x, iptr_pad):
    mesh = plsc.VectorSubcoreMesh(core_axis_name="c", subcore_axis_name="s")
    f = functools.partial(
        pl.kernel,
        mesh=mesh,
        out_type=jax.ShapeDtypeStruct((S, D), jnp.float32),
        scratch_types=[
            pltpu.VMEM((IPTR_BUF,), jnp.int32),
            pltpu.VMEM((2, CHUNK, D), jnp.float32),
            pltpu.VMEM((SEG_W, D), jnp.float32),
            pltpu.SemaphoreType.DMA,
            pltpu.SemaphoreType.DMA,
        ],
    )(_seg_kernel)
    return f(x, iptr_pad)


def kernel(x, indptr):
    pad = jnp.full((IPTR_PAD - (S + 1),), N, dtype=indptr.dtype)
    iptr_pad = jnp.concatenate([indptr, pad])
    return _run(x, iptr_pad)


# speedup vs baseline: 27.5329x; 1.0343x over previous
import functools

import jax
import jax.numpy as jnp
from jax import lax
from jax.experimental import pallas as pl
from jax.experimental.pallas import tpu as pltpu
from jax.experimental.pallas import tpu_sc as plsc

N = 320000
S = 10000
D = 128
NLANE = 16
NVEC = D // NLANE

NW = 32
SEG_W = 320
SEG_LAST = S - (NW - 1) * SEG_W
IPTR_BUF = 344
IPTR_PAD = (NW - 1) * SEG_W + IPTR_BUF
CHUNK = 256


def _ld(ref, i):
    return ref[pl.ds(i, NLANE)][0]


HALF = CHUNK // 2


def _seg_kernel(x_hbm, iptr_hbm, out_hbm, iptr_v, buf, out_v,
                sem0a, sem0b, sem1a, sem1b):
    wid = lax.axis_index("s") * 2 + lax.axis_index("c")
    s0 = pl.multiple_of(wid * SEG_W, 8)

    pltpu.sync_copy(iptr_hbm.at[pl.ds(s0, IPTR_BUF)], iptr_v)

    zero = jnp.zeros((NLANE,), jnp.float32)

    nseg = jnp.where(wid == NW - 1, SEG_LAST, SEG_W)
    r0 = _ld(iptr_v, 0)
    r_end = _ld(iptr_v, nseg)
    base0 = (r0 // 8) * 8
    n_chunks = (r_end - base0 + CHUNK - 1) // CHUNK

    def start_of(win):
        return pl.multiple_of(
            jnp.minimum(base0 + win * CHUNK, N - CHUNK), 8)

    def _issue(w_start, b, sa, sb):
        h2 = pl.multiple_of(w_start + HALF, 8)
        pltpu.async_copy(x_hbm.at[pl.ds(w_start, HALF)],
                         buf.at[b, pl.ds(0, HALF)], sa)
        pltpu.async_copy(x_hbm.at[pl.ds(h2, HALF)],
                         buf.at[b, pl.ds(HALF, HALF)], sb)

    def _wait(w_start, b, sa, sb):
        h2 = pl.multiple_of(w_start + HALF, 8)
        pltpu.make_async_copy(x_hbm.at[pl.ds(w_start, HALF)],
                              buf.at[b, pl.ds(0, HALF)], sa).wait()
        pltpu.make_async_copy(x_hbm.at[pl.ds(h2, HALF)],
                              buf.at[b, pl.ds(HALF, HALF)], sb).wait()

    @pl.when(n_chunks > 0)
    def _prime():
        _issue(start_of(0), 0, sem0a, sem0b)

    def body(t, carry):
        r_cur, s, staged_win, *acc = carry
        win = (r_cur - base0) // CHUNK
        start = start_of(win)
        win_end = base0 + (win + 1) * CHUNK
        p = win % 2

        @pl.when(jnp.logical_and(win != staged_win, win < n_chunks))
        def _advance():
            @pl.when(p == 0)
            def _():
                _wait(start, 0, sem0a, sem0b)

            @pl.when(p == 1)
            def _():
                _wait(start, 1, sem1a, sem1b)

            @pl.when(win + 1 < n_chunks)
            def _prefetch():
                nstart = start_of(win + 1)

                @pl.when(p == 0)
                def _():
                    _issue(nstart, 1, sem1a, sem1b)

                @pl.when(p == 1)
                def _():
                    _issue(nstart, 0, sem0a, sem0b)

        seg_end = _ld(iptr_v, s + 1)
        e = jnp.minimum(jnp.minimum(seg_end, win_end), r_end)
        n_rows = e - r_cur

        def row_quad(k, ac):
            idx = r_cur + 4 * k - start
            ac = list(ac)
            for u in range(4):
                for j in range(NVEC):
                    ac[j] = ac[j] + buf[p, idx + u, pl.ds(j * NLANE, NLANE)]
            return tuple(ac)

        def row_one(i, ac):
            idx = r_cur + i - start
            return tuple(
                ac[j] + buf[p, idx, pl.ds(j * NLANE, NLANE)]
                for j in range(NVEC)
            )

        acc = lax.fori_loop(0, n_rows // 4, row_quad, tuple(acc))
        acc = lax.fori_loop(n_rows // 4 * 4, n_rows, row_one, acc)

        finished = jnp.logical_and(e >= seg_end, s < nseg)

        @pl.when(finished)
        def _flush():
            for j in range(NVEC):
                out_v[s, pl.ds(j * NLANE, NLANE)] = acc[j]

        s_next = s + jnp.where(finished, 1, 0)
        acc_next = tuple(jnp.where(finished, zero, a) for a in acc)
        return (e, s_next, win) + acc_next

    init = (r0, jnp.int32(0), jnp.int32(-1)) + tuple(zero for _ in range(NVEC))
    lax.fori_loop(0, n_chunks + SEG_W, body, init)

    @pl.when(wid < NW - 1)
    def _store_full():
        pltpu.sync_copy(out_v, out_hbm.at[pl.ds(s0, SEG_W)])

    @pl.when(wid == NW - 1)
    def _store_last():
        pltpu.sync_copy(out_v.at[pl.ds(0, SEG_LAST)],
                        out_hbm.at[pl.ds(s0, SEG_LAST)])


@jax.jit
def _run(x, iptr_pad):
    mesh = plsc.VectorSubcoreMesh(core_axis_name="c", subcore_axis_name="s")
    f = functools.partial(
        pl.kernel,
        mesh=mesh,
        out_type=jax.ShapeDtypeStruct((S, D), jnp.float32),
        scratch_types=[
            pltpu.VMEM((IPTR_BUF,), jnp.int32),
            pltpu.VMEM((2, CHUNK, D), jnp.float32),
            pltpu.VMEM((SEG_W, D), jnp.float32),
            pltpu.SemaphoreType.DMA,
            pltpu.SemaphoreType.DMA,
            pltpu.SemaphoreType.DMA,
            pltpu.SemaphoreType.DMA,
        ],
    )(_seg_kernel)
    return f(x, iptr_pad)


def kernel(x, indptr):
    pad = jnp.full((IPTR_PAD - (S + 1),), N, dtype=indptr.dtype)
    iptr_pad = jnp.concatenate([indptr, pad])
    return _run(x, iptr_pad)


# speedup vs baseline: 28.3589x; 1.0300x over previous
import functools

import jax
import jax.numpy as jnp
from jax import lax
from jax.experimental import pallas as pl
from jax.experimental.pallas import tpu as pltpu
from jax.experimental.pallas import tpu_sc as plsc

N = 320000
S = 10000
D = 128
NLANE = 16
NVEC = D // NLANE

NW = 32
SEG_W = 320
SEG_LAST = S - (NW - 1) * SEG_W
IPTR_BUF = 344
IPTR_PAD = (NW - 1) * SEG_W + IPTR_BUF
CHUNK = 192
NBUF = 3


def _ld(ref, i):
    return ref[pl.ds(i, NLANE)][0]


HALF = CHUNK // 2


def _seg_kernel(x_hbm, iptr_hbm, out_hbm, iptr_v, buf, out_v,
                s0a, s0b, s1a, s1b, s2a, s2b):
    sems = ((s0a, s0b), (s1a, s1b), (s2a, s2b))
    wid = lax.axis_index("s") * 2 + lax.axis_index("c")
    s0 = pl.multiple_of(wid * SEG_W, 8)

    pltpu.sync_copy(iptr_hbm.at[pl.ds(s0, IPTR_BUF)], iptr_v)

    zero = jnp.zeros((NLANE,), jnp.float32)

    nseg = jnp.where(wid == NW - 1, SEG_LAST, SEG_W)
    r0 = _ld(iptr_v, 0)
    r_end = _ld(iptr_v, nseg)
    base0 = (r0 // 8) * 8
    n_chunks = (r_end - base0 + CHUNK - 1) // CHUNK

    def start_of(win):
        return pl.multiple_of(
            jnp.minimum(base0 + win * CHUNK, N - CHUNK), 8)

    def _issue(w_start, b):
        sa, sb = sems[b]
        h2 = pl.multiple_of(w_start + HALF, 8)
        pltpu.async_copy(x_hbm.at[pl.ds(w_start, HALF)],
                         buf.at[b, pl.ds(0, HALF)], sa)
        pltpu.async_copy(x_hbm.at[pl.ds(h2, HALF)],
                         buf.at[b, pl.ds(HALF, HALF)], sb)

    def _wait(w_start, b):
        sa, sb = sems[b]
        h2 = pl.multiple_of(w_start + HALF, 8)
        pltpu.make_async_copy(x_hbm.at[pl.ds(w_start, HALF)],
                              buf.at[b, pl.ds(0, HALF)], sa).wait()
        pltpu.make_async_copy(x_hbm.at[pl.ds(h2, HALF)],
                              buf.at[b, pl.ds(HALF, HALF)], sb).wait()

    @pl.when(n_chunks > 0)
    def _prime0():
        _issue(start_of(0), 0)

    @pl.when(n_chunks > 1)
    def _prime1():
        _issue(start_of(1), 1)

    def body(t, carry):
        r_cur, s, staged_win, *acc = carry
        win = (r_cur - base0) // CHUNK
        start = start_of(win)
        win_end = base0 + (win + 1) * CHUNK
        p = win % NBUF

        @pl.when(jnp.logical_and(win != staged_win, win < n_chunks))
        def _advance():
            for q in range(NBUF):
                @pl.when(p == q)
                def _(q=q):
                    _wait(start, q)

            @pl.when(win + 2 < n_chunks)
            def _prefetch():
                nstart = start_of(win + 2)
                t = (win + 2) % NBUF
                for q in range(NBUF):
                    @pl.when(t == q)
                    def _(q=q):
                        _issue(nstart, q)

        seg_end = _ld(iptr_v, s + 1)
        e = jnp.minimum(jnp.minimum(seg_end, win_end), r_end)
        n_rows = e - r_cur

        def row_oct(k, ac):
            idx = r_cur + 8 * k - start
            ac = list(ac)
            for u in range(8):
                for j in range(NVEC):
                    ac[j] = ac[j] + buf[p, idx + u, pl.ds(j * NLANE, NLANE)]
            return tuple(ac)

        def row_one(i, ac):
            idx = r_cur + i - start
            return tuple(
                ac[j] + buf[p, idx, pl.ds(j * NLANE, NLANE)]
                for j in range(NVEC)
            )

        acc = lax.fori_loop(0, n_rows // 8, row_oct, tuple(acc))
        acc = lax.fori_loop(n_rows // 8 * 8, n_rows, row_one, acc)

        finished = jnp.logical_and(e >= seg_end, s < nseg)

        @pl.when(finished)
        def _flush():
            for j in range(NVEC):
                out_v[s, pl.ds(j * NLANE, NLANE)] = acc[j]

        s_next = s + jnp.where(finished, 1, 0)
        acc_next = tuple(jnp.where(finished, zero, a) for a in acc)
        return (e, s_next, win) + acc_next

    init = (r0, jnp.int32(0), jnp.int32(-1)) + tuple(zero for _ in range(NVEC))
    lax.fori_loop(0, n_chunks + SEG_W, body, init)

    @pl.when(wid < NW - 1)
    def _store_full():
        pltpu.sync_copy(out_v, out_hbm.at[pl.ds(s0, SEG_W)])

    @pl.when(wid == NW - 1)
    def _store_last():
        pltpu.sync_copy(out_v.at[pl.ds(0, SEG_LAST)],
                        out_hbm.at[pl.ds(s0, SEG_LAST)])


@jax.jit
def _run(x, iptr_pad):
    mesh = plsc.VectorSubcoreMesh(core_axis_name="c", subcore_axis_name="s")
    f = functools.partial(
        pl.kernel,
        mesh=mesh,
        out_type=jax.ShapeDtypeStruct((S, D), jnp.float32),
        scratch_types=[
            pltpu.VMEM((IPTR_BUF,), jnp.int32),
            pltpu.VMEM((NBUF, CHUNK, D), jnp.float32),
            pltpu.VMEM((SEG_W, D), jnp.float32),
            pltpu.SemaphoreType.DMA,
            pltpu.SemaphoreType.DMA,
            pltpu.SemaphoreType.DMA,
            pltpu.SemaphoreType.DMA,
            pltpu.SemaphoreType.DMA,
            pltpu.SemaphoreType.DMA,
        ],
    )(_seg_kernel)
    return f(x, iptr_pad)


def kernel(x, indptr):
    pad = jnp.full((IPTR_PAD - (S + 1),), N, dtype=indptr.dtype)
    iptr_pad = jnp.concatenate([indptr, pad])
    return _run(x, iptr_pad)


# speedup vs baseline: 29.4559x; 1.0387x over previous
import functools

import jax
import jax.numpy as jnp
from jax import lax
from jax.experimental import pallas as pl
from jax.experimental.pallas import tpu as pltpu
from jax.experimental.pallas import tpu_sc as plsc

N = 320000
S = 10000
D = 128
NLANE = 16
NVEC = D // NLANE

NW = 32
SEG_W = 320
SEG_LAST = S - (NW - 1) * SEG_W
IPTR_BUF = 344
IPTR_PAD = (NW - 1) * SEG_W + IPTR_BUF
CHUNK = 192
NBUF = 3


def _ld(ref, i):
    return ref[pl.ds(i, NLANE)][0]


HALF = CHUNK // 2


def _seg_kernel(x_hbm, iptr_hbm, out_hbm, iptr_v, buf, out_v,
                s0a, s0b, s1a, s1b, s2a, s2b):
    sems = ((s0a, s0b), (s1a, s1b), (s2a, s2b))
    wid = lax.axis_index("s") * 2 + lax.axis_index("c")
    s0 = pl.multiple_of(wid * SEG_W, 8)

    pltpu.sync_copy(iptr_hbm.at[pl.ds(s0, IPTR_BUF)], iptr_v)

    zero = jnp.zeros((NLANE,), jnp.float32)

    nseg = jnp.where(wid == NW - 1, SEG_LAST, SEG_W)
    r0 = _ld(iptr_v, 0)
    r_end = _ld(iptr_v, nseg)
    base0 = (r0 // 8) * 8
    n_chunks = (r_end - base0 + CHUNK - 1) // CHUNK

    def start_of(win):
        return pl.multiple_of(
            jnp.minimum(base0 + win * CHUNK, N - CHUNK), 8)

    def _issue(w_start, b):
        sa, sb = sems[b]
        h2 = pl.multiple_of(w_start + HALF, 8)
        pltpu.async_copy(x_hbm.at[pl.ds(w_start, HALF)],
                         buf.at[b, pl.ds(0, HALF)], sa)
        pltpu.async_copy(x_hbm.at[pl.ds(h2, HALF)],
                         buf.at[b, pl.ds(HALF, HALF)], sb)

    def _wait(w_start, b):
        sa, sb = sems[b]
        h2 = pl.multiple_of(w_start + HALF, 8)
        pltpu.make_async_copy(x_hbm.at[pl.ds(w_start, HALF)],
                              buf.at[b, pl.ds(0, HALF)], sa).wait()
        pltpu.make_async_copy(x_hbm.at[pl.ds(h2, HALF)],
                              buf.at[b, pl.ds(HALF, HALF)], sb).wait()

    @pl.when(n_chunks > 0)
    def _prime0():
        _issue(start_of(0), 0)

    @pl.when(n_chunks > 1)
    def _prime1():
        _issue(start_of(1), 1)

    def body(t, carry):
        r_cur, s, win_prev, end_prev, p_prev, *acc = carry
        adv = jnp.where(r_cur >= end_prev, 1, 0)
        win = win_prev + adv
        win_end = end_prev + adv * CHUNK
        p = jnp.where(adv == 1,
                      jnp.where(p_prev == NBUF - 1, 0, p_prev + 1),
                      p_prev)
        start = pl.multiple_of(jnp.minimum(win_end - CHUNK, N - CHUNK), 8)

        @pl.when(jnp.logical_and(adv == 1, win < n_chunks))
        def _advance():
            for q in range(NBUF):
                @pl.when(p == q)
                def _(q=q):
                    _wait(start, q)

            @pl.when(win + 2 < n_chunks)
            def _prefetch():
                nstart = start_of(win + 2)
                for q in range(NBUF):
                    @pl.when((p + 2) % NBUF == q)
                    def _(q=q):
                        _issue(nstart, q)

        seg_end = _ld(iptr_v, s + 1)
        e = jnp.minimum(jnp.minimum(seg_end, win_end), r_end)
        n_rows = e - r_cur

        def row_oct(k, ac):
            idx = r_cur + 8 * k - start
            ac = list(ac)
            for u in range(8):
                for j in range(NVEC):
                    ac[j] = ac[j] + buf[p, idx + u, pl.ds(j * NLANE, NLANE)]
            return tuple(ac)

        def row_one(i, ac):
            idx = r_cur + i - start
            return tuple(
                ac[j] + buf[p, idx, pl.ds(j * NLANE, NLANE)]
                for j in range(NVEC)
            )

        acc = lax.fori_loop(0, n_rows // 8, row_oct, tuple(acc))
        acc = lax.fori_loop(n_rows // 8 * 8, n_rows, row_one, acc)

        finished = jnp.logical_and(e >= seg_end, s < nseg)

        @pl.when(finished)
        def _flush():
            for j in range(NVEC):
                out_v[s, pl.ds(j * NLANE, NLANE)] = acc[j]

        s_next = s + jnp.where(finished, 1, 0)
        acc_next = tuple(jnp.where(finished, zero, a) for a in acc)
        return (e, s_next, win, win_end, p) + acc_next

    init = (r0, jnp.int32(0), jnp.int32(-1), base0, jnp.int32(-1)) \
        + tuple(zero for _ in range(NVEC))
    lax.fori_loop(0, n_chunks + SEG_W, body, init)

    @pl.when(wid < NW - 1)
    def _store_full():
        pltpu.sync_copy(out_v, out_hbm.at[pl.ds(s0, SEG_W)])

    @pl.when(wid == NW - 1)
    def _store_last():
        pltpu.sync_copy(out_v.at[pl.ds(0, SEG_LAST)],
                        out_hbm.at[pl.ds(s0, SEG_LAST)])


@jax.jit
def _run(x, iptr_pad):
    mesh = plsc.VectorSubcoreMesh(core_axis_name="c", subcore_axis_name="s")
    f = functools.partial(
        pl.kernel,
        mesh=mesh,
        out_type=jax.ShapeDtypeStruct((S, D), jnp.float32),
        scratch_types=[
            pltpu.VMEM((IPTR_BUF,), jnp.int32),
            pltpu.VMEM((NBUF, CHUNK, D), jnp.float32),
            pltpu.VMEM((SEG_W, D), jnp.float32),
            pltpu.SemaphoreType.DMA,
            pltpu.SemaphoreType.DMA,
            pltpu.SemaphoreType.DMA,
            pltpu.SemaphoreType.DMA,
            pltpu.SemaphoreType.DMA,
            pltpu.SemaphoreType.DMA,
        ],
    )(_seg_kernel)
    return f(x, iptr_pad)


def kernel(x, indptr):
    pad = jnp.full((IPTR_PAD - (S + 1),), N, dtype=indptr.dtype)
    iptr_pad = jnp.concatenate([indptr, pad])
    return _run(x, iptr_pad)


# speedup vs baseline: 29.5954x; 1.0047x over previous
import functools

import jax
import jax.numpy as jnp
from jax import lax
from jax.experimental import pallas as pl
from jax.experimental.pallas import tpu as pltpu
from jax.experimental.pallas import tpu_sc as plsc

N = 320000
S = 10000
D = 128
NLANE = 16
NVEC = D // NLANE

NW = 32
SEG_W = 320
SEG_LAST = S - (NW - 1) * SEG_W
IPTR_BUF = 344
IPTR_PAD = (NW - 1) * SEG_W + IPTR_BUF
CHUNK = 224
NBUF = 3
NSPLIT = 4


def _ld(ref, i):
    return ref[pl.ds(i, NLANE)][0]


PART = CHUNK // NSPLIT


def _seg_kernel(x_hbm, iptr_hbm, out_hbm, iptr_v, buf, out_v, *sems_flat):
    sems = tuple(sems_flat[b * NSPLIT:(b + 1) * NSPLIT] for b in range(NBUF))
    wid = lax.axis_index("s") * 2 + lax.axis_index("c")
    s0 = pl.multiple_of(wid * SEG_W, 8)

    pltpu.sync_copy(iptr_hbm.at[pl.ds(s0, IPTR_BUF)], iptr_v)

    zero = jnp.zeros((NLANE,), jnp.float32)

    nseg = jnp.where(wid == NW - 1, SEG_LAST, SEG_W)
    r0 = _ld(iptr_v, 0)
    r_end = _ld(iptr_v, nseg)
    base0 = (r0 // 8) * 8
    n_chunks = (r_end - base0 + CHUNK - 1) // CHUNK

    def start_of(win):
        return pl.multiple_of(
            jnp.minimum(base0 + win * CHUNK, N - CHUNK), 8)

    def _issue(w_start, b):
        for u in range(NSPLIT):
            off = pl.multiple_of(w_start + u * PART, 8)
            pltpu.async_copy(x_hbm.at[pl.ds(off, PART)],
                             buf.at[b, pl.ds(u * PART, PART)], sems[b][u])

    def _wait(w_start, b):
        for u in range(NSPLIT):
            off = pl.multiple_of(w_start + u * PART, 8)
            pltpu.make_async_copy(x_hbm.at[pl.ds(off, PART)],
                                  buf.at[b, pl.ds(u * PART, PART)],
                                  sems[b][u]).wait()

    @pl.when(n_chunks > 0)
    def _prime0():
        _issue(start_of(0), 0)

    @pl.when(n_chunks > 1)
    def _prime1():
        _issue(start_of(1), 1)

    def body(t, carry):
        r_cur, s, win_prev, end_prev, p_prev, *acc = carry
        adv = jnp.where(r_cur >= end_prev, 1, 0)
        win = win_prev + adv
        win_end = end_prev + adv * CHUNK
        p = jnp.where(adv == 1,
                      jnp.where(p_prev == NBUF - 1, 0, p_prev + 1),
                      p_prev)
        start = pl.multiple_of(jnp.minimum(win_end - CHUNK, N - CHUNK), 8)

        @pl.when(jnp.logical_and(adv == 1, win < n_chunks))
        def _advance():
            for q in range(NBUF):
                @pl.when(p == q)
                def _(q=q):
                    _wait(start, q)

            @pl.when(win + 2 < n_chunks)
            def _prefetch():
                nstart = start_of(win + 2)
                for q in range(NBUF):
                    @pl.when((p + 2) % NBUF == q)
                    def _(q=q):
                        _issue(nstart, q)

        seg_end = _ld(iptr_v, s + 1)
        e = jnp.minimum(jnp.minimum(seg_end, win_end), r_end)
        n_rows = e - r_cur

        def row_oct(k, ac):
            idx = r_cur + 8 * k - start
            ac = list(ac)
            for u in range(8):
                for j in range(NVEC):
                    ac[j] = ac[j] + buf[p, idx + u, pl.ds(j * NLANE, NLANE)]
            return tuple(ac)

        def row_one(i, ac):
            idx = r_cur + i - start
            return tuple(
                ac[j] + buf[p, idx, pl.ds(j * NLANE, NLANE)]
                for j in range(NVEC)
            )

        acc = lax.fori_loop(0, n_rows // 8, row_oct, tuple(acc))
        acc = lax.fori_loop(n_rows // 8 * 8, n_rows, row_one, acc)

        finished = jnp.logical_and(e >= seg_end, s < nseg)

        @pl.when(finished)
        def _flush():
            for j in range(NVEC):
                out_v[s, pl.ds(j * NLANE, NLANE)] = acc[j]

        s_next = s + jnp.where(finished, 1, 0)
        acc_next = tuple(jnp.where(finished, zero, a) for a in acc)
        return (e, s_next, win, win_end, p) + acc_next

    init = (r0, jnp.int32(0), jnp.int32(-1), base0, jnp.int32(-1)) \
        + tuple(zero for _ in range(NVEC))
    lax.fori_loop(0, n_chunks + SEG_W, body, init)

    @pl.when(wid < NW - 1)
    def _store_full():
        pltpu.sync_copy(out_v, out_hbm.at[pl.ds(s0, SEG_W)])

    @pl.when(wid == NW - 1)
    def _store_last():
        pltpu.sync_copy(out_v.at[pl.ds(0, SEG_LAST)],
                        out_hbm.at[pl.ds(s0, SEG_LAST)])


@jax.jit
def _run(x, iptr_pad):
    mesh = plsc.VectorSubcoreMesh(core_axis_name="c", subcore_axis_name="s")
    f = functools.partial(
        pl.kernel,
        mesh=mesh,
        out_type=jax.ShapeDtypeStruct((S, D), jnp.float32),
        scratch_types=[
            pltpu.VMEM((IPTR_BUF,), jnp.int32),
            pltpu.VMEM((NBUF, CHUNK, D), jnp.float32),
            pltpu.VMEM((SEG_W, D), jnp.float32),
        ] + [pltpu.SemaphoreType.DMA] * (NBUF * NSPLIT),
    )(_seg_kernel)
    return f(x, iptr_pad)


def kernel(x, indptr):
    pad = jnp.full((IPTR_PAD - (S + 1),), N, dtype=indptr.dtype)
    iptr_pad = jnp.concatenate([indptr, pad])
    return _run(x, iptr_pad)
